# baseline (speedup 1.0000x reference)
import functools

import jax
import jax.numpy as jnp
from jax import lax
from jax.experimental import pallas as pl
from jax.experimental.pallas import tpu as pltpu
from jax.experimental.pallas import tpu_sc as plsc

_N = 10000
_NPAD = 10112
_B = 64
_ZROWS = 64
_CAPR = 192


def _sc_compact(erows, nc, ns):
    nw = nc * ns
    ert = erows // nw
    ept = ert * 128
    ng = -(-ert // _B)
    mesh = plsc.VectorSubcoreMesh(core_axis_name="c", subcore_axis_name="s")

    def body(src_hbm, dst_hbm, et_hbm,
             c2s_hbm, c2d_hbm, c3s_hbm, c3d_hbm, cn2_hbm, cn3_hbm,
             eidx, tbuf, sbuf, dbuf, l2s, l2d, l3s, l3d, cbuf, sem):
        cid = lax.axis_index("c")
        sid = lax.axis_index("s")
        wid = cid * ns + sid

        for k in range(ng * 4):
            q = k * 16 + lax.iota(jnp.int32, 16)
            v = jnp.where(q < ert, wid * ert + q, wid * ert)
            eidx[k // 4, pl.ds((k % 4) * 16, 16)] = v
        for g in range(ng):
            pltpu.async_copy(et_hbm.at[eidx.at[g]],
                             tbuf.at[pl.ds(g * _B, _B)], sem).wait()
            pltpu.async_copy(src_hbm.at[eidx.at[g]],
                             sbuf.at[pl.ds(g * _B, _B)], sem).wait()
            pltpu.async_copy(dst_hbm.at[eidx.at[g]],
                             dbuf.at[pl.ds(g * _B, _B)], sem).wait()

        def cbody(i, c):
            cnt2, cnt3 = c
            r = i // 8
            j = (i % 8) * 16
            t16 = tbuf[r, pl.ds(j, 16)]
            s16 = sbuf[r, pl.ds(j, 16)]
            d16 = dbuf[r, pl.ds(j, 16)]
            outs = []
            for rel, cnt, ls_, ld_ in ((2, cnt2, l2s, l2d),
                                       (3, cnt3, l3s, l3d)):
                m = t16 == rel
                wi = jnp.where(m, jnp.int32(1), jnp.int32(0))
                pos = plsc.cumsum(wi)
                idx = cnt + pos - 1
                ridx = lax.shift_right_arithmetic(idx, 6)
                cidx = lax.bitwise_and(idx, _B - 1)
                plsc.store_scatter(ls_, [ridx, cidx], s16, mask=m)
                plsc.store_scatter(ld_, [ridx, cidx], d16, mask=m)
                outs.append(cnt + pos[15])
            return tuple(outs)

        cnt2, cnt3 = lax.fori_loop(0, ept // 16, cbody,
                                   (jnp.int32(0), jnp.int32(0)))

        for cnt, ls_, ld_ in ((cnt2, l2s, l2d), (cnt3, l3s, l3d)):
            for j in range(4 * _B // 16):
                idx = cnt + j * 16 + lax.iota(jnp.int32, 16)
                ridx = lax.shift_right_arithmetic(idx, 6)
                cidx = lax.bitwise_and(idx, _B - 1)
                plsc.store_scatter(ls_, [ridx, cidx],
                                   jnp.zeros((16,), jnp.int32),
                                   mask=jnp.full((16,), True))
                plsc.store_scatter(ld_, [ridx, cidx],
                                   jnp.full((16,), _N, jnp.int32),
                                   mask=jnp.full((16,), True))

        for cnt, cn_hbm in ((cnt2, cn2_hbm), (cnt3, cn3_hbm)):
            cbuf[pl.ds(0, 16)] = cnt + jnp.zeros((16,), jnp.int32)
            pltpu.sync_copy(cbuf, cn_hbm.at[wid])

        row0 = wid * _CAPR
        pltpu.sync_copy(l2s, c2s_hbm.at[pl.ds(row0, _CAPR)])
        pltpu.sync_copy(l2d, c2d_hbm.at[pl.ds(row0, _CAPR)])
        pltpu.sync_copy(l3s, c3s_hbm.at[pl.ds(row0, _CAPR)])
        pltpu.sync_copy(l3d, c3d_hbm.at[pl.ds(row0, _CAPR)])

    ltype = jax.ShapeDtypeStruct((nw * _CAPR, _B), jnp.int32)
    ctype = jax.ShapeDtypeStruct((nw, 16), jnp.int32)
    return pl.kernel(
        body,
        out_type=[ltype, ltype, ltype, ltype, ctype, ctype],
        mesh=mesh,
        compiler_params=pltpu.CompilerParams(
            needs_layout_passes=False, use_tc_tiling_on_sc=False),
        scratch_types=[
            pltpu.VMEM((ng, _B), jnp.int32),
            pltpu.VMEM((ng * _B, 128), jnp.int32),
            pltpu.VMEM((ng * _B, 128), jnp.int32),
            pltpu.VMEM((ng * _B, 128), jnp.int32),
            pltpu.VMEM((_CAPR, _B), jnp.int32),
            pltpu.VMEM((_CAPR, _B), jnp.int32),
            pltpu.VMEM((_CAPR, _B), jnp.int32),
            pltpu.VMEM((_CAPR, _B), jnp.int32),
            pltpu.VMEM((16,), jnp.int32),
            pltpu.SemaphoreType.DMA,
        ],
    )


def _sc_pass(n_pad, d, nc, ns):
    nw = nc * ns
    rows_pt = n_pad // ns
    mesh = plsc.VectorSubcoreMesh(core_axis_name="c", subcore_axis_name="s")

    def body(h_hbm, cls_hbm, cld_hbm, cnt_hbm, out_hbm,
             lidx, sbuf, dbuf, cbuf, rows_a, rows_b, zbuf, acc,
             sem, sem_a, sem_b):
        cid = lax.axis_index("c")
        sid = lax.axis_index("s")
        wid = cid * ns + sid
        row0 = sid * rows_pt

        for k in range((_CAPR // _B) * 4):
            lidx[k // 4, pl.ds((k % 4) * 16, 16)] = (
                wid * _CAPR + k * 16 + lax.iota(jnp.int32, 16))
        stage = []
        for g in range(_CAPR // _B):
            stage.append(pltpu.async_copy(
                cls_hbm.at[lidx.at[g]], sbuf.at[pl.ds(g * _B, _B)], sem))
            stage.append(pltpu.async_copy(
                cld_hbm.at[lidx.at[g]], dbuf.at[pl.ds(g * _B, _B)], sem))

        def zfill(r, carry):
            for j in range(d // 16):
                zbuf[r, pl.ds(j * 16, 16)] = jnp.zeros((16,), jnp.float32)
            return carry
        lax.fori_loop(0, _ZROWS, zfill, 0)
        for t in range(rows_pt // _ZROWS):
            pltpu.sync_copy(zbuf, acc.at[pl.ds(row0 + t * _ZROWS, _ZROWS)])
        zrem = rows_pt % _ZROWS
        if zrem:
            pltpu.sync_copy(
                zbuf.at[pl.ds(0, zrem)],
                acc.at[pl.ds(row0 + rows_pt - zrem, zrem)])

        for c in stage:
            c.wait()
        pltpu.sync_copy(cnt_hbm.at[wid], cbuf)
        cnt = cbuf[pl.ds(0, 16)][0]

        plsc.subcore_barrier()

        nb2 = (cnt + (2 * _B - 1)) // (2 * _B)
        pltpu.async_copy(h_hbm.at[sbuf.at[0]], rows_a, sem_a)
        pltpu.async_copy(h_hbm.at[sbuf.at[1]], rows_b, sem_b)

        def pair(i, carry):
            pltpu.make_async_copy(h_hbm.at[sbuf.at[2 * i]],
                                  rows_a, sem_a).wait()
            pltpu.sync_copy(rows_a, acc.at[dbuf.at[2 * i]], add=True)
            pltpu.async_copy(h_hbm.at[sbuf.at[2 * i + 2]], rows_a, sem_a)
            pltpu.make_async_copy(h_hbm.at[sbuf.at[2 * i + 1]],
                                  rows_b, sem_b).wait()
            pltpu.sync_copy(rows_b, acc.at[dbuf.at[2 * i + 1]], add=True)
            pltpu.async_copy(h_hbm.at[sbuf.at[2 * i + 3]], rows_b, sem_b)
            return carry

        lax.fori_loop(0, nb2, pair, 0)
        pltpu.make_async_copy(h_hbm.at[sbuf.at[0]], rows_a, sem_a).wait()
        pltpu.make_async_copy(h_hbm.at[sbuf.at[1]], rows_b, sem_b).wait()

        plsc.subcore_barrier()

        pltpu.sync_copy(acc.at[pl.ds(row0, rows_pt)],
                        out_hbm.at[cid, pl.ds(row0, rows_pt)])

    return pl.kernel(
        body,
        out_type=jax.ShapeDtypeStruct((nc, n_pad, d), jnp.float32),
        mesh=mesh,
        compiler_params=pltpu.CompilerParams(
            needs_layout_passes=False, use_tc_tiling_on_sc=False),
        scratch_types=[
            pltpu.VMEM((_CAPR // _B, _B), jnp.int32),
            pltpu.VMEM((_CAPR, _B), jnp.int32),
            pltpu.VMEM((_CAPR, _B), jnp.int32),
            pltpu.VMEM((16,), jnp.int32),
            pltpu.VMEM((_B, d), jnp.float32),
            pltpu.VMEM((_B, d), jnp.float32),
            pltpu.VMEM((_ZROWS, d), jnp.float32),
            pltpu.VMEM_SHARED((n_pad, d), jnp.float32),
            pltpu.SemaphoreType.DMA,
            pltpu.SemaphoreType.DMA,
            pltpu.SemaphoreType.DMA,
        ],
    )


def _bcast(shape):
    return pl.BlockSpec(shape, lambda i: tuple(0 for _ in shape))


_R = 2000


def _tc1(x, w1, b1, w2, b2, w3, b3, lw0, lb0, lw1, lb1):
    n, d_in = x.shape
    hd = w1.shape[1]
    d2 = w3.shape[1]

    def body(x_r, w1_r, b1_r, w2_r, b2_r, w3_r, b3_r, lw0_r, lb0_r, lw1_r,
             lb1_r, h_r, hw0_r):
        h = jnp.maximum(
            jnp.dot(x_r[...], w1_r[...], preferred_element_type=jnp.float32)
            + b1_r[...], 0.0)
        h = jnp.maximum(
            jnp.dot(h, w2_r[...], preferred_element_type=jnp.float32)
            + b2_r[...], 0.0)
        h = (jnp.dot(h, w3_r[...], preferred_element_type=jnp.float32)
             + b3_r[...])
        h_r[...] = h
        hw0_r[...] = (jnp.dot(h, lw0_r[...] + lw1_r[...],
                              preferred_element_type=jnp.float32)
                      + lb0_r[...] + lb1_r[...])

    return pl.pallas_call(
        body,
        grid=(n // _R,),
        in_specs=[
            pl.BlockSpec((_R, d_in), lambda i: (i, 0)),
            _bcast((d_in, hd)), _bcast((1, hd)),
            _bcast((hd, hd)), _bcast((1, hd)),
            _bcast((hd, d2)), _bcast((1, d2)),
            _bcast((d2, hd)), _bcast((1, hd)),
            _bcast((d2, hd)), _bcast((1, hd)),
        ],
        out_specs=[
            pl.BlockSpec((_R, d2), lambda i: (i, 0)),
            pl.BlockSpec((_R, hd), lambda i: (i, 0)),
        ],
        out_shape=[
            jax.ShapeDtypeStruct((n, d2), jnp.float32),
            jax.ShapeDtypeStruct((n, hd), jnp.float32),
        ],
    )(x, w1, b1, w2, b2, w3, b3, lw0, lb0, lw1, lb1)


def _tc2(p0, hw0, wl, bl, w0, b0, w1, b1):
    nc, n_pad, d2 = p0.shape
    n, hd = hw0.shape

    def body(p_r, hw0_r, wl_r, bl_r, w0_r, b0_r, w1_r, b1_r, emb_r, e1w_r):
        aggr = p_r[0] + p_r[1]
        emb = jnp.maximum(
            jnp.dot(aggr, wl_r[...], preferred_element_type=jnp.float32)
            + bl_r[...] + hw0_r[...], 0.0)
        emb_r[...] = emb
        e1w_r[...] = (jnp.dot(emb, w0_r[...] + w1_r[...],
                              preferred_element_type=jnp.float32)
                      + b0_r[...] + b1_r[...])

    return pl.pallas_call(
        body,
        grid=(n // _R,),
        in_specs=[
            pl.BlockSpec((nc, _R, d2), lambda i: (0, i, 0)),
            pl.BlockSpec((_R, hd), lambda i: (i, 0)),
            _bcast((d2, hd)), _bcast((1, hd)),
            _bcast((hd, hd)), _bcast((1, hd)),
            _bcast((hd, hd)), _bcast((1, hd)),
        ],
        out_specs=[
            pl.BlockSpec((_R, hd), lambda i: (i, 0)),
            pl.BlockSpec((_R, hd), lambda i: (i, 0)),
        ],
        out_shape=[
            jax.ShapeDtypeStruct((n, hd), jnp.float32),
            jax.ShapeDtypeStruct((n, hd), jnp.float32),
        ],
    )(p0, hw0, wl, bl, w0, b0, w1, b1)


def _tc3(p1, e1w, wl, bl, fc1_w, fc1_b, fc2_w, fc2_b):
    nc, n_pad, hd = p1.shape
    n = e1w.shape[0]
    d_out = fc2_w.shape[1]

    def body(p_r, e1w_r, wl_r, bl_r, fc1w_r, fc1b_r, fc2w_r, fc2b_r, o_r):
        aggr = p_r[0] + p_r[1]
        emb = jnp.maximum(
            jnp.dot(aggr, wl_r[...], preferred_element_type=jnp.float32)
            + bl_r[...] + e1w_r[...], 0.0)
        fc1c = fc1w_r[0:hd, :] + fc1w_r[hd:2 * hd, :]
        t = jnp.maximum(
            jnp.dot(emb, fc1c, preferred_element_type=jnp.float32)
            + fc1b_r[...], 0.0)
        o = (jnp.dot(t, fc2w_r[...], preferred_element_type=jnp.float32)
             + fc2b_r[...])
        m = jnp.max(o, axis=1, keepdims=True)
        z = o - m
        o_r[...] = z - jnp.log(jnp.sum(jnp.exp(z), axis=1, keepdims=True))

    return pl.pallas_call(
        body,
        grid=(n // _R,),
        in_specs=[
            pl.BlockSpec((nc, _R, hd), lambda i: (0, i, 0)),
            pl.BlockSpec((_R, hd), lambda i: (i, 0)),
            _bcast((hd, hd)), _bcast((1, hd)),
            _bcast((2 * hd, hd)), _bcast((1, hd)),
            _bcast((hd, d_out)), _bcast((1, d_out)),
        ],
        out_specs=pl.BlockSpec((_R, d_out), lambda i: (i, 0)),
        out_shape=jax.ShapeDtypeStruct((n, d_out), jnp.float32),
    )(p1, e1w, wl, bl, fc1_w, fc1_b, fc2_w, fc2_b)


def kernel(x, edge_index, edge_type,
           mlp_w1, mlp_b1, mlp_w2, mlp_b2, mlp_w3, mlp_b3,
           l0_w0, l0_b0, l0_wl, l0_bl, l0_w1, l0_b1,
           l1_w0, l1_b0, l1_wl, l1_bl, l1_w1, l1_b1,
           fc1_w, fc1_b, fc2_w, fc2_b):
    e = edge_type.shape[0]
    info = plsc.get_sparse_core_info()
    nc, ns = info.num_cores, info.num_subcores
    nw = nc * ns

    erows = -(-e // (128 * nw)) * nw
    epad = erows * 128 - e
    se = jnp.pad(edge_index[1], (0, epad)).reshape(erows, 128)
    de = jnp.pad(edge_index[0], (0, epad)).reshape(erows, 128)
    te = jnp.pad(edge_type, (0, epad),
                 constant_values=-1).reshape(erows, 128)

    r2 = lambda b: b.reshape(1, -1)

    cl2s, cl2d, cl3s, cl3d, cn2, cn3 = _sc_compact(erows, nc, ns)(se, de, te)
    h, hw0 = _tc1(x, mlp_w1, r2(mlp_b1), mlp_w2, r2(mlp_b2),
                  mlp_w3, r2(mlp_b3), l0_w0, r2(l0_b0), l0_w1, r2(l0_b1))

    p0 = _sc_pass(_NPAD, h.shape[1], nc, ns)(h, cl2s, cl2d, cn2)
    emb1, e1w = _tc2(p0, hw0, l0_wl, r2(l0_bl),
                     l1_w0, r2(l1_b0), l1_w1, r2(l1_b1))
    p1 = _sc_pass(_NPAD, emb1.shape[1], nc, ns)(emb1, cl3s, cl3d, cn3)
    return _tc3(p1, e1w, l1_wl, r2(l1_bl), fc1_w, r2(fc1_b),
                fc2_w, r2(fc2_b))


# speedup vs baseline: 1.7352x; 1.7352x over previous
import functools

import jax
import jax.numpy as jnp
from jax import lax
from jax.experimental import pallas as pl
from jax.experimental.pallas import tpu as pltpu
from jax.experimental.pallas import tpu_sc as plsc

_N = 10000
_NPAD = 10112
_B = 64
_ZROWS = 64
_CAPR = 192


def _sc_compact(erows, nc, ns):
    nw = nc * ns
    ert = erows // nw
    ept = ert * 128
    ng = -(-ert // _B)
    mesh = plsc.VectorSubcoreMesh(core_axis_name="c", subcore_axis_name="s")

    def body(src_hbm, dst_hbm, et_hbm,
             c2s_hbm, c2d_hbm, c3s_hbm, c3d_hbm, cn2_hbm, cn3_hbm,
             eidx, tbuf, sbuf, dbuf, l2s, l2d, l3s, l3d, cbuf, sem):
        cid = lax.axis_index("c")
        sid = lax.axis_index("s")
        wid = cid * ns + sid

        for k in range(ng * 4):
            q = k * 16 + lax.iota(jnp.int32, 16)
            v = jnp.where(q < ert, wid * ert + q, wid * ert)
            eidx[k // 4, pl.ds((k % 4) * 16, 16)] = v
        for g in range(ng):
            pltpu.async_copy(et_hbm.at[eidx.at[g]],
                             tbuf.at[pl.ds(g * _B, _B)], sem).wait()
            pltpu.async_copy(src_hbm.at[eidx.at[g]],
                             sbuf.at[pl.ds(g * _B, _B)], sem).wait()
            pltpu.async_copy(dst_hbm.at[eidx.at[g]],
                             dbuf.at[pl.ds(g * _B, _B)], sem).wait()

        def cbody(i, c):
            cnt2, cnt3 = c
            r = i // 8
            j = (i % 8) * 16
            t16 = tbuf[r, pl.ds(j, 16)]
            s16 = sbuf[r, pl.ds(j, 16)]
            d16 = dbuf[r, pl.ds(j, 16)]
            outs = []
            for rel, cnt, ls_, ld_ in ((2, cnt2, l2s, l2d),
                                       (3, cnt3, l3s, l3d)):
                m = t16 == rel
                wi = jnp.where(m, jnp.int32(1), jnp.int32(0))
                pos = plsc.cumsum(wi)
                idx = cnt + pos - 1
                ridx = lax.shift_right_arithmetic(idx, 6)
                cidx = lax.bitwise_and(idx, _B - 1)
                plsc.store_scatter(ls_, [ridx, cidx], s16, mask=m)
                plsc.store_scatter(ld_, [ridx, cidx], d16, mask=m)
                outs.append(cnt + pos[15])
            return tuple(outs)

        cnt2, cnt3 = lax.fori_loop(0, ept // 16, cbody,
                                   (jnp.int32(0), jnp.int32(0)))

        for cnt, ls_, ld_ in ((cnt2, l2s, l2d), (cnt3, l3s, l3d)):
            for j in range(4 * _B // 16):
                idx = cnt + j * 16 + lax.iota(jnp.int32, 16)
                ridx = lax.shift_right_arithmetic(idx, 6)
                cidx = lax.bitwise_and(idx, _B - 1)
                plsc.store_scatter(ls_, [ridx, cidx],
                                   jnp.zeros((16,), jnp.int32),
                                   mask=jnp.full((16,), True))
                plsc.store_scatter(ld_, [ridx, cidx],
                                   jnp.full((16,), _N, jnp.int32),
                                   mask=jnp.full((16,), True))

        for cnt, cn_hbm in ((cnt2, cn2_hbm), (cnt3, cn3_hbm)):
            cbuf[pl.ds(0, 16)] = cnt + jnp.zeros((16,), jnp.int32)
            pltpu.sync_copy(cbuf, cn_hbm.at[wid])

        row0 = wid * _CAPR
        pltpu.sync_copy(l2s, c2s_hbm.at[pl.ds(row0, _CAPR)])
        pltpu.sync_copy(l2d, c2d_hbm.at[pl.ds(row0, _CAPR)])
        pltpu.sync_copy(l3s, c3s_hbm.at[pl.ds(row0, _CAPR)])
        pltpu.sync_copy(l3d, c3d_hbm.at[pl.ds(row0, _CAPR)])

    ltype = jax.ShapeDtypeStruct((nw * _CAPR, _B), jnp.int32)
    ctype = jax.ShapeDtypeStruct((nw, 16), jnp.int32)
    return pl.kernel(
        body,
        out_type=[ltype, ltype, ltype, ltype, ctype, ctype],
        mesh=mesh,
        compiler_params=pltpu.CompilerParams(
            needs_layout_passes=False, use_tc_tiling_on_sc=False),
        scratch_types=[
            pltpu.VMEM((ng, _B), jnp.int32),
            pltpu.VMEM((ng * _B, 128), jnp.int32),
            pltpu.VMEM((ng * _B, 128), jnp.int32),
            pltpu.VMEM((ng * _B, 128), jnp.int32),
            pltpu.VMEM((_CAPR, _B), jnp.int32),
            pltpu.VMEM((_CAPR, _B), jnp.int32),
            pltpu.VMEM((_CAPR, _B), jnp.int32),
            pltpu.VMEM((_CAPR, _B), jnp.int32),
            pltpu.VMEM((16,), jnp.int32),
            pltpu.SemaphoreType.DMA,
        ],
    )


def _sc_pass(n_pad, d, nc, ns):
    nw = nc * ns
    rows_pt = n_pad // ns
    mesh = plsc.VectorSubcoreMesh(core_axis_name="c", subcore_axis_name="s")

    def body(h_hbm, cls_hbm, cld_hbm, cnt_hbm, out_hbm,
             lidx, sbuf, dbuf, cbuf, rows_a, rows_b, zbuf, acc,
             sem, sem_a, sem_b):
        cid = lax.axis_index("c")
        sid = lax.axis_index("s")
        wid = cid * ns + sid
        row0 = sid * rows_pt

        for k in range((_CAPR // _B) * 4):
            lidx[k // 4, pl.ds((k % 4) * 16, 16)] = (
                wid * _CAPR + k * 16 + lax.iota(jnp.int32, 16))
        stage = []
        for g in range(_CAPR // _B):
            stage.append(pltpu.async_copy(
                cls_hbm.at[lidx.at[g]], sbuf.at[pl.ds(g * _B, _B)], sem))
            stage.append(pltpu.async_copy(
                cld_hbm.at[lidx.at[g]], dbuf.at[pl.ds(g * _B, _B)], sem))

        def zfill(r, carry):
            for j in range(d // 16):
                zbuf[r, pl.ds(j * 16, 16)] = jnp.zeros((16,), jnp.float32)
            return carry
        lax.fori_loop(0, _ZROWS, zfill, 0)
        for t in range(rows_pt // _ZROWS):
            pltpu.sync_copy(zbuf, acc.at[pl.ds(row0 + t * _ZROWS, _ZROWS)])
        zrem = rows_pt % _ZROWS
        if zrem:
            pltpu.sync_copy(
                zbuf.at[pl.ds(0, zrem)],
                acc.at[pl.ds(row0 + rows_pt - zrem, zrem)])

        for c in stage:
            c.wait()
        pltpu.sync_copy(cnt_hbm.at[wid], cbuf)
        cnt = cbuf[pl.ds(0, 16)][0]

        plsc.subcore_barrier()

        nb2 = (cnt + (2 * _B - 1)) // (2 * _B)

        def pair(i, carry):
            ca = pltpu.async_copy(h_hbm.at[sbuf.at[2 * i]], rows_a, sem_a)
            cb = pltpu.async_copy(h_hbm.at[sbuf.at[2 * i + 1]], rows_b, sem_b)
            ca.wait()
            pltpu.sync_copy(rows_a, acc.at[dbuf.at[2 * i]], add=True)
            cb.wait()
            pltpu.sync_copy(rows_b, acc.at[dbuf.at[2 * i + 1]], add=True)
            return carry

        lax.fori_loop(0, nb2, pair, 0)

        plsc.subcore_barrier()

        pltpu.sync_copy(acc.at[pl.ds(row0, rows_pt)],
                        out_hbm.at[cid, pl.ds(row0, rows_pt)])

    return pl.kernel(
        body,
        out_type=jax.ShapeDtypeStruct((nc, n_pad, d), jnp.float32),
        mesh=mesh,
        compiler_params=pltpu.CompilerParams(
            needs_layout_passes=False, use_tc_tiling_on_sc=False),
        scratch_types=[
            pltpu.VMEM((_CAPR // _B, _B), jnp.int32),
            pltpu.VMEM((_CAPR, _B), jnp.int32),
            pltpu.VMEM((_CAPR, _B), jnp.int32),
            pltpu.VMEM((16,), jnp.int32),
            pltpu.VMEM((_B, d), jnp.float32),
            pltpu.VMEM((_B, d), jnp.float32),
            pltpu.VMEM((_ZROWS, d), jnp.float32),
            pltpu.VMEM_SHARED((n_pad, d), jnp.float32),
            pltpu.SemaphoreType.DMA,
            pltpu.SemaphoreType.DMA,
            pltpu.SemaphoreType.DMA,
        ],
    )


def _bcast(shape):
    return pl.BlockSpec(shape, lambda i: tuple(0 for _ in shape))


_R = 2000


def _tc1(x, w1, b1, w2, b2, w3, b3, lw0, lb0, lw1, lb1):
    n, d_in = x.shape
    hd = w1.shape[1]
    d2 = w3.shape[1]

    def body(x_r, w1_r, b1_r, w2_r, b2_r, w3_r, b3_r, lw0_r, lb0_r, lw1_r,
             lb1_r, h_r, hw0_r):
        h = jnp.maximum(
            jnp.dot(x_r[...], w1_r[...], preferred_element_type=jnp.float32)
            + b1_r[...], 0.0)
        h = jnp.maximum(
            jnp.dot(h, w2_r[...], preferred_element_type=jnp.float32)
            + b2_r[...], 0.0)
        h = (jnp.dot(h, w3_r[...], preferred_element_type=jnp.float32)
             + b3_r[...])
        h_r[...] = h
        hw0_r[...] = (jnp.dot(h, lw0_r[...] + lw1_r[...],
                              preferred_element_type=jnp.float32)
                      + lb0_r[...] + lb1_r[...])

    return pl.pallas_call(
        body,
        grid=(n // _R,),
        in_specs=[
            pl.BlockSpec((_R, d_in), lambda i: (i, 0)),
            _bcast((d_in, hd)), _bcast((1, hd)),
            _bcast((hd, hd)), _bcast((1, hd)),
            _bcast((hd, d2)), _bcast((1, d2)),
            _bcast((d2, hd)), _bcast((1, hd)),
            _bcast((d2, hd)), _bcast((1, hd)),
        ],
        out_specs=[
            pl.BlockSpec((_R, d2), lambda i: (i, 0)),
            pl.BlockSpec((_R, hd), lambda i: (i, 0)),
        ],
        out_shape=[
            jax.ShapeDtypeStruct((n, d2), jnp.float32),
            jax.ShapeDtypeStruct((n, hd), jnp.float32),
        ],
    )(x, w1, b1, w2, b2, w3, b3, lw0, lb0, lw1, lb1)


def _tc2(p0, hw0, wl, bl, w0, b0, w1, b1):
    nc, n_pad, d2 = p0.shape
    n, hd = hw0.shape

    def body(p_r, hw0_r, wl_r, bl_r, w0_r, b0_r, w1_r, b1_r, emb_r, e1w_r):
        aggr = p_r[0] + p_r[1]
        emb = jnp.maximum(
            jnp.dot(aggr, wl_r[...], preferred_element_type=jnp.float32)
            + bl_r[...] + hw0_r[...], 0.0)
        emb_r[...] = emb
        e1w_r[...] = (jnp.dot(emb, w0_r[...] + w1_r[...],
                              preferred_element_type=jnp.float32)
                      + b0_r[...] + b1_r[...])

    return pl.pallas_call(
        body,
        grid=(n // _R,),
        in_specs=[
            pl.BlockSpec((nc, _R, d2), lambda i: (0, i, 0)),
            pl.BlockSpec((_R, hd), lambda i: (i, 0)),
            _bcast((d2, hd)), _bcast((1, hd)),
            _bcast((hd, hd)), _bcast((1, hd)),
            _bcast((hd, hd)), _bcast((1, hd)),
        ],
        out_specs=[
            pl.BlockSpec((_R, hd), lambda i: (i, 0)),
            pl.BlockSpec((_R, hd), lambda i: (i, 0)),
        ],
        out_shape=[
            jax.ShapeDtypeStruct((n, hd), jnp.float32),
            jax.ShapeDtypeStruct((n, hd), jnp.float32),
        ],
    )(p0, hw0, wl, bl, w0, b0, w1, b1)


def _tc3(p1, e1w, wl, bl, fc1_w, fc1_b, fc2_w, fc2_b):
    nc, n_pad, hd = p1.shape
    n = e1w.shape[0]
    d_out = fc2_w.shape[1]

    def body(p_r, e1w_r, wl_r, bl_r, fc1w_r, fc1b_r, fc2w_r, fc2b_r, o_r):
        aggr = p_r[0] + p_r[1]
        emb = jnp.maximum(
            jnp.dot(aggr, wl_r[...], preferred_element_type=jnp.float32)
            + bl_r[...] + e1w_r[...], 0.0)
        fc1c = fc1w_r[0:hd, :] + fc1w_r[hd:2 * hd, :]
        t = jnp.maximum(
            jnp.dot(emb, fc1c, preferred_element_type=jnp.float32)
            + fc1b_r[...], 0.0)
        o = (jnp.dot(t, fc2w_r[...], preferred_element_type=jnp.float32)
             + fc2b_r[...])
        m = jnp.max(o, axis=1, keepdims=True)
        z = o - m
        o_r[...] = z - jnp.log(jnp.sum(jnp.exp(z), axis=1, keepdims=True))

    return pl.pallas_call(
        body,
        grid=(n // _R,),
        in_specs=[
            pl.BlockSpec((nc, _R, hd), lambda i: (0, i, 0)),
            pl.BlockSpec((_R, hd), lambda i: (i, 0)),
            _bcast((hd, hd)), _bcast((1, hd)),
            _bcast((2 * hd, hd)), _bcast((1, hd)),
            _bcast((hd, d_out)), _bcast((1, d_out)),
        ],
        out_specs=pl.BlockSpec((_R, d_out), lambda i: (i, 0)),
        out_shape=jax.ShapeDtypeStruct((n, d_out), jnp.float32),
    )(p1, e1w, wl, bl, fc1_w, fc1_b, fc2_w, fc2_b)


def kernel(x, edge_index, edge_type,
           mlp_w1, mlp_b1, mlp_w2, mlp_b2, mlp_w3, mlp_b3,
           l0_w0, l0_b0, l0_wl, l0_bl, l0_w1, l0_b1,
           l1_w0, l1_b0, l1_wl, l1_bl, l1_w1, l1_b1,
           fc1_w, fc1_b, fc2_w, fc2_b):
    e = edge_type.shape[0]
    info = plsc.get_sparse_core_info()
    nc, ns = info.num_cores, info.num_subcores
    nw = nc * ns

    erows = -(-e // (128 * nw)) * nw
    epad = erows * 128 - e
    se = jnp.pad(edge_index[1], (0, epad)).reshape(erows, 128)
    de = jnp.pad(edge_index[0], (0, epad)).reshape(erows, 128)
    te = jnp.pad(edge_type, (0, epad),
                 constant_values=-1).reshape(erows, 128)

    r2 = lambda b: b.reshape(1, -1)

    cl2s, cl2d, cl3s, cl3d, cn2, cn3 = _sc_compact(erows, nc, ns)(se, de, te)
    h, hw0 = _tc1(x, mlp_w1, r2(mlp_b1), mlp_w2, r2(mlp_b2),
                  mlp_w3, r2(mlp_b3), l0_w0, r2(l0_b0), l0_w1, r2(l0_b1))

    p0 = _sc_pass(_NPAD, h.shape[1], nc, ns)(h, cl2s, cl2d, cn2)
    emb1, e1w = _tc2(p0, hw0, l0_wl, r2(l0_bl),
                     l1_w0, r2(l1_b0), l1_w1, r2(l1_b1))
    p1 = _sc_pass(_NPAD, emb1.shape[1], nc, ns)(emb1, cl3s, cl3d, cn3)
    return _tc3(p1, e1w, l1_wl, r2(l1_bl), fc1_w, r2(fc1_b),
                fc2_w, r2(fc2_b))


# speedup vs baseline: 1.7427x; 1.0043x over previous
import functools

import jax
import jax.numpy as jnp
from jax import lax
from jax.experimental import pallas as pl
from jax.experimental.pallas import tpu as pltpu
from jax.experimental.pallas import tpu_sc as plsc

_N = 10000
_NPAD = 10112
_B = 64
_LW = 128
_ZROWS = 64
_CAPR = 96


def _sc_compact(erows, nc, ns):
    nw = nc * ns
    ert = erows // nw
    ept = ert * 128
    ng = -(-ert // _B)
    mesh = plsc.VectorSubcoreMesh(core_axis_name="c", subcore_axis_name="s")

    def body(src_hbm, dst_hbm, et_hbm,
             c2s_hbm, c2d_hbm, c3s_hbm, c3d_hbm, cn2_hbm, cn3_hbm,
             eidx, tbuf, sbuf, dbuf, l2s, l2d, l3s, l3d, cbuf, sem):
        cid = lax.axis_index("c")
        sid = lax.axis_index("s")
        wid = cid * ns + sid

        for k in range(ng * 4):
            q = k * 16 + lax.iota(jnp.int32, 16)
            v = jnp.where(q < ert, wid * ert + q, wid * ert)
            eidx[k // 4, pl.ds((k % 4) * 16, 16)] = v
        for g in range(ng):
            pltpu.async_copy(et_hbm.at[eidx.at[g]],
                             tbuf.at[pl.ds(g * _B, _B)], sem).wait()
            pltpu.async_copy(src_hbm.at[eidx.at[g]],
                             sbuf.at[pl.ds(g * _B, _B)], sem).wait()
            pltpu.async_copy(dst_hbm.at[eidx.at[g]],
                             dbuf.at[pl.ds(g * _B, _B)], sem).wait()

        def cbody(i, c):
            cnt2, cnt3 = c
            r = i // 8
            j = (i % 8) * 16
            t16 = tbuf[r, pl.ds(j, 16)]
            s16 = sbuf[r, pl.ds(j, 16)]
            d16 = dbuf[r, pl.ds(j, 16)]
            outs = []
            for rel, cnt, ls_, ld_ in ((2, cnt2, l2s, l2d),
                                       (3, cnt3, l3s, l3d)):
                m = t16 == rel
                wi = jnp.where(m, jnp.int32(1), jnp.int32(0))
                pos = plsc.cumsum(wi)
                idx = cnt + pos - 1
                ridx = lax.shift_right_arithmetic(idx, 7)
                cidx = lax.bitwise_and(idx, _LW - 1)
                plsc.store_scatter(ls_, [ridx, cidx], s16, mask=m)
                plsc.store_scatter(ld_, [ridx, cidx], d16, mask=m)
                outs.append(cnt + pos[15])
            return tuple(outs)

        cnt2, cnt3 = lax.fori_loop(0, ept // 16, cbody,
                                   (jnp.int32(0), jnp.int32(0)))

        for cnt, ls_, ld_ in ((cnt2, l2s, l2d), (cnt3, l3s, l3d)):
            for j in range(_LW // 16):
                idx = cnt + j * 16 + lax.iota(jnp.int32, 16)
                ridx = lax.shift_right_arithmetic(idx, 7)
                cidx = lax.bitwise_and(idx, _LW - 1)
                plsc.store_scatter(ls_, [ridx, cidx],
                                   jnp.zeros((16,), jnp.int32),
                                   mask=jnp.full((16,), True))
                plsc.store_scatter(ld_, [ridx, cidx],
                                   jnp.full((16,), _N, jnp.int32),
                                   mask=jnp.full((16,), True))

        for cnt, cn_hbm in ((cnt2, cn2_hbm), (cnt3, cn3_hbm)):
            cbuf[pl.ds(0, 16)] = cnt + jnp.zeros((16,), jnp.int32)
            pltpu.sync_copy(cbuf, cn_hbm.at[wid])

        row0 = wid * _CAPR
        pltpu.sync_copy(l2s, c2s_hbm.at[pl.ds(row0, _CAPR)])
        pltpu.sync_copy(l2d, c2d_hbm.at[pl.ds(row0, _CAPR)])
        pltpu.sync_copy(l3s, c3s_hbm.at[pl.ds(row0, _CAPR)])
        pltpu.sync_copy(l3d, c3d_hbm.at[pl.ds(row0, _CAPR)])

    ltype = jax.ShapeDtypeStruct((nw * _CAPR, _LW), jnp.int32)
    ctype = jax.ShapeDtypeStruct((nw, 16), jnp.int32)
    return pl.kernel(
        body,
        out_type=[ltype, ltype, ltype, ltype, ctype, ctype],
        mesh=mesh,
        compiler_params=pltpu.CompilerParams(
            needs_layout_passes=False, use_tc_tiling_on_sc=False),
        scratch_types=[
            pltpu.VMEM((ng, _B), jnp.int32),
            pltpu.VMEM((ng * _B, 128), jnp.int32),
            pltpu.VMEM((ng * _B, 128), jnp.int32),
            pltpu.VMEM((ng * _B, 128), jnp.int32),
            pltpu.VMEM((_CAPR, _LW), jnp.int32),
            pltpu.VMEM((_CAPR, _LW), jnp.int32),
            pltpu.VMEM((_CAPR, _LW), jnp.int32),
            pltpu.VMEM((_CAPR, _LW), jnp.int32),
            pltpu.VMEM((16,), jnp.int32),
            pltpu.SemaphoreType.DMA,
        ],
    )


def _sc_pass(n_pad, d, nc, ns):
    nw = nc * ns
    rows_pt = n_pad // ns
    mesh = plsc.VectorSubcoreMesh(core_axis_name="c", subcore_axis_name="s")

    def body(h_hbm, cls_hbm, cld_hbm, cnt_hbm, out_hbm,
             lidx, lidxb, sbuf, dbuf, cbuf, rows_ab, zbuf, acc,
             sem, sem_a, sem_b):
        rows_a = rows_ab.at[pl.ds(0, _B)]
        rows_b = rows_ab.at[pl.ds(_B, _B)]
        cid = lax.axis_index("c")
        sid = lax.axis_index("s")
        wid = cid * ns + sid
        row0 = sid * rows_pt

        for k in range(4):
            lidx[pl.ds(k * 16, 16)] = (wid * _CAPR + k * 16
                                       + lax.iota(jnp.int32, 16))
        for k in range(2):
            lidxb[pl.ds(k * 16, 16)] = (wid * _CAPR + _B + k * 16
                                        + lax.iota(jnp.int32, 16))
        stage = [
            pltpu.async_copy(cls_hbm.at[lidx], sbuf.at[pl.ds(0, _B)], sem),
            pltpu.async_copy(cld_hbm.at[lidx], dbuf.at[pl.ds(0, _B)], sem),
            pltpu.async_copy(cls_hbm.at[lidxb],
                             sbuf.at[pl.ds(_B, _CAPR - _B)], sem),
            pltpu.async_copy(cld_hbm.at[lidxb],
                             dbuf.at[pl.ds(_B, _CAPR - _B)], sem),
        ]

        def zfill(r, carry):
            for j in range(d // 16):
                zbuf[r, pl.ds(j * 16, 16)] = jnp.zeros((16,), jnp.float32)
            return carry
        lax.fori_loop(0, _ZROWS, zfill, 0)
        for t in range(rows_pt // _ZROWS):
            pltpu.sync_copy(zbuf, acc.at[pl.ds(row0 + t * _ZROWS, _ZROWS)])
        zrem = rows_pt % _ZROWS
        if zrem:
            pltpu.sync_copy(
                zbuf.at[pl.ds(0, zrem)],
                acc.at[pl.ds(row0 + rows_pt - zrem, zrem)])

        for c in stage:
            c.wait()
        pltpu.sync_copy(cnt_hbm.at[wid], cbuf)
        cnt = cbuf[pl.ds(0, 16)][0]

        plsc.subcore_barrier()

        nb = (cnt + (_LW - 1)) // _LW

        def gs(i, carry):
            ca = pltpu.async_copy(h_hbm.at[sbuf.at[i, pl.ds(0, _B)]],
                                  rows_a, sem_a)
            cb = pltpu.async_copy(h_hbm.at[sbuf.at[i, pl.ds(_B, _B)]],
                                  rows_b, sem_b)
            ca.wait()
            cb.wait()
            pltpu.sync_copy(rows_ab, acc.at[dbuf.at[i]], add=True)
            return carry

        lax.fori_loop(0, nb, gs, 0)

        plsc.subcore_barrier()

        pltpu.sync_copy(acc.at[pl.ds(row0, rows_pt)],
                        out_hbm.at[cid, pl.ds(row0, rows_pt)])

    return pl.kernel(
        body,
        out_type=jax.ShapeDtypeStruct((nc, n_pad, d), jnp.float32),
        mesh=mesh,
        compiler_params=pltpu.CompilerParams(
            needs_layout_passes=False, use_tc_tiling_on_sc=False),
        scratch_types=[
            pltpu.VMEM((_B,), jnp.int32),
            pltpu.VMEM((_CAPR - _B,), jnp.int32),
            pltpu.VMEM((_CAPR, _LW), jnp.int32),
            pltpu.VMEM((_CAPR, _LW), jnp.int32),
            pltpu.VMEM((16,), jnp.int32),
            pltpu.VMEM((2 * _B, d), jnp.float32),
            pltpu.VMEM((_ZROWS, d), jnp.float32),
            pltpu.VMEM_SHARED((n_pad, d), jnp.float32),
            pltpu.SemaphoreType.DMA,
            pltpu.SemaphoreType.DMA,
            pltpu.SemaphoreType.DMA,
        ],
    )


def _bcast(shape):
    return pl.BlockSpec(shape, lambda i: tuple(0 for _ in shape))


_R = 2000


def _tc1(x, w1, b1, w2, b2, w3, b3, lw0, lb0, lw1, lb1):
    n, d_in = x.shape
    hd = w1.shape[1]
    d2 = w3.shape[1]

    def body(x_r, w1_r, b1_r, w2_r, b2_r, w3_r, b3_r, lw0_r, lb0_r, lw1_r,
             lb1_r, h_r, hw0_r):
        h = jnp.maximum(
            jnp.dot(x_r[...], w1_r[...], preferred_element_type=jnp.float32)
            + b1_r[...], 0.0)
        h = jnp.maximum(
            jnp.dot(h, w2_r[...], preferred_element_type=jnp.float32)
            + b2_r[...], 0.0)
        h = (jnp.dot(h, w3_r[...], preferred_element_type=jnp.float32)
             + b3_r[...])
        h_r[...] = h
        hw0_r[...] = (jnp.dot(h, lw0_r[...] + lw1_r[...],
                              preferred_element_type=jnp.float32)
                      + lb0_r[...] + lb1_r[...])

    return pl.pallas_call(
        body,
        grid=(n // _R,),
        in_specs=[
            pl.BlockSpec((_R, d_in), lambda i: (i, 0)),
            _bcast((d_in, hd)), _bcast((1, hd)),
            _bcast((hd, hd)), _bcast((1, hd)),
            _bcast((hd, d2)), _bcast((1, d2)),
            _bcast((d2, hd)), _bcast((1, hd)),
            _bcast((d2, hd)), _bcast((1, hd)),
        ],
        out_specs=[
            pl.BlockSpec((_R, d2), lambda i: (i, 0)),
            pl.BlockSpec((_R, hd), lambda i: (i, 0)),
        ],
        out_shape=[
            jax.ShapeDtypeStruct((n, d2), jnp.float32),
            jax.ShapeDtypeStruct((n, hd), jnp.float32),
        ],
    )(x, w1, b1, w2, b2, w3, b3, lw0, lb0, lw1, lb1)


def _tc2(p0, hw0, wl, bl, w0, b0, w1, b1):
    nc, n_pad, d2 = p0.shape
    n, hd = hw0.shape

    def body(p_r, hw0_r, wl_r, bl_r, w0_r, b0_r, w1_r, b1_r, emb_r, e1w_r):
        aggr = p_r[0] + p_r[1]
        emb = jnp.maximum(
            jnp.dot(aggr, wl_r[...], preferred_element_type=jnp.float32)
            + bl_r[...] + hw0_r[...], 0.0)
        emb_r[...] = emb
        e1w_r[...] = (jnp.dot(emb, w0_r[...] + w1_r[...],
                              preferred_element_type=jnp.float32)
                      + b0_r[...] + b1_r[...])

    return pl.pallas_call(
        body,
        grid=(n // _R,),
        in_specs=[
            pl.BlockSpec((nc, _R, d2), lambda i: (0, i, 0)),
            pl.BlockSpec((_R, hd), lambda i: (i, 0)),
            _bcast((d2, hd)), _bcast((1, hd)),
            _bcast((hd, hd)), _bcast((1, hd)),
            _bcast((hd, hd)), _bcast((1, hd)),
        ],
        out_specs=[
            pl.BlockSpec((_R, hd), lambda i: (i, 0)),
            pl.BlockSpec((_R, hd), lambda i: (i, 0)),
        ],
        out_shape=[
            jax.ShapeDtypeStruct((n, hd), jnp.float32),
            jax.ShapeDtypeStruct((n, hd), jnp.float32),
        ],
    )(p0, hw0, wl, bl, w0, b0, w1, b1)


def _tc3(p1, e1w, wl, bl, fc1_w, fc1_b, fc2_w, fc2_b):
    nc, n_pad, hd = p1.shape
    n = e1w.shape[0]
    d_out = fc2_w.shape[1]

    def body(p_r, e1w_r, wl_r, bl_r, fc1w_r, fc1b_r, fc2w_r, fc2b_r, o_r):
        aggr = p_r[0] + p_r[1]
        emb = jnp.maximum(
            jnp.dot(aggr, wl_r[...], preferred_element_type=jnp.float32)
            + bl_r[...] + e1w_r[...], 0.0)
        fc1c = fc1w_r[0:hd, :] + fc1w_r[hd:2 * hd, :]
        t = jnp.maximum(
            jnp.dot(emb, fc1c, preferred_element_type=jnp.float32)
            + fc1b_r[...], 0.0)
        o = (jnp.dot(t, fc2w_r[...], preferred_element_type=jnp.float32)
             + fc2b_r[...])
        m = jnp.max(o, axis=1, keepdims=True)
        z = o - m
        o_r[...] = z - jnp.log(jnp.sum(jnp.exp(z), axis=1, keepdims=True))

    return pl.pallas_call(
        body,
        grid=(n // _R,),
        in_specs=[
            pl.BlockSpec((nc, _R, hd), lambda i: (0, i, 0)),
            pl.BlockSpec((_R, hd), lambda i: (i, 0)),
            _bcast((hd, hd)), _bcast((1, hd)),
            _bcast((2 * hd, hd)), _bcast((1, hd)),
            _bcast((hd, d_out)), _bcast((1, d_out)),
        ],
        out_specs=pl.BlockSpec((_R, d_out), lambda i: (i, 0)),
        out_shape=jax.ShapeDtypeStruct((n, d_out), jnp.float32),
    )(p1, e1w, wl, bl, fc1_w, fc1_b, fc2_w, fc2_b)


def kernel(x, edge_index, edge_type,
           mlp_w1, mlp_b1, mlp_w2, mlp_b2, mlp_w3, mlp_b3,
           l0_w0, l0_b0, l0_wl, l0_bl, l0_w1, l0_b1,
           l1_w0, l1_b0, l1_wl, l1_bl, l1_w1, l1_b1,
           fc1_w, fc1_b, fc2_w, fc2_b):
    e = edge_type.shape[0]
    info = plsc.get_sparse_core_info()
    nc, ns = info.num_cores, info.num_subcores
    nw = nc * ns

    erows = -(-e // (128 * nw)) * nw
    epad = erows * 128 - e
    se = jnp.pad(edge_index[1], (0, epad)).reshape(erows, 128)
    de = jnp.pad(edge_index[0], (0, epad)).reshape(erows, 128)
    te = jnp.pad(edge_type, (0, epad),
                 constant_values=-1).reshape(erows, 128)

    r2 = lambda b: b.reshape(1, -1)

    cl2s, cl2d, cl3s, cl3d, cn2, cn3 = _sc_compact(erows, nc, ns)(se, de, te)
    h, hw0 = _tc1(x, mlp_w1, r2(mlp_b1), mlp_w2, r2(mlp_b2),
                  mlp_w3, r2(mlp_b3), l0_w0, r2(l0_b0), l0_w1, r2(l0_b1))

    p0 = _sc_pass(_NPAD, h.shape[1], nc, ns)(h, cl2s, cl2d, cn2)
    emb1, e1w = _tc2(p0, hw0, l0_wl, r2(l0_bl),
                     l1_w0, r2(l1_b0), l1_w1, r2(l1_b1))
    p1 = _sc_pass(_NPAD, emb1.shape[1], nc, ns)(emb1, cl3s, cl3d, cn3)
    return _tc3(p1, e1w, l1_wl, r2(l1_bl), fc1_w, r2(fc1_b),
                fc2_w, r2(fc2_b))


# speedup vs baseline: 1.8508x; 1.0620x over previous
import functools

import jax
import jax.numpy as jnp
from jax import lax
from jax.experimental import pallas as pl
from jax.experimental.pallas import tpu as pltpu
from jax.experimental.pallas import tpu_sc as plsc

_N = 10000
_NPAD = 10112
_B = 64
_ZROWS = 64
_CAPR = 192


def _sc_compact(erows, nc, ns):
    nw = nc * ns
    ert = erows // nw
    ept = ert * 128
    ng = -(-ert // _B)
    mesh = plsc.VectorSubcoreMesh(core_axis_name="c", subcore_axis_name="s")

    def body(src_hbm, dst_hbm, et_hbm,
             c2s_hbm, c2d_hbm, c3s_hbm, c3d_hbm, cn2_hbm, cn3_hbm,
             eidx, tbuf, sbuf, dbuf, l2s, l2d, l3s, l3d, cbuf, sem):
        cid = lax.axis_index("c")
        sid = lax.axis_index("s")
        wid = cid * ns + sid

        for k in range(ng * 4):
            q = k * 16 + lax.iota(jnp.int32, 16)
            v = jnp.where(q < ert, wid * ert + q, wid * ert)
            eidx[k // 4, pl.ds((k % 4) * 16, 16)] = v
        for g in range(ng):
            pltpu.async_copy(et_hbm.at[eidx.at[g]],
                             tbuf.at[pl.ds(g * _B, _B)], sem).wait()
            pltpu.async_copy(src_hbm.at[eidx.at[g]],
                             sbuf.at[pl.ds(g * _B, _B)], sem).wait()
            pltpu.async_copy(dst_hbm.at[eidx.at[g]],
                             dbuf.at[pl.ds(g * _B, _B)], sem).wait()

        def cbody(i, c):
            cnt2, cnt3 = c
            r = i // 8
            j = (i % 8) * 16
            t16 = tbuf[r, pl.ds(j, 16)]
            s16 = sbuf[r, pl.ds(j, 16)]
            d16 = dbuf[r, pl.ds(j, 16)]
            outs = []
            for rel, cnt, ls_, ld_ in ((2, cnt2, l2s, l2d),
                                       (3, cnt3, l3s, l3d)):
                m = t16 == rel
                wi = jnp.where(m, jnp.int32(1), jnp.int32(0))
                pos = plsc.cumsum(wi)
                idx = cnt + pos - 1
                ridx = lax.shift_right_arithmetic(idx, 6)
                cidx = lax.bitwise_and(idx, _B - 1)
                plsc.store_scatter(ls_, [ridx, cidx], s16, mask=m)
                plsc.store_scatter(ld_, [ridx, cidx], d16, mask=m)
                outs.append(cnt + pos[15])
            return tuple(outs)

        cnt2, cnt3 = lax.fori_loop(0, ept // 16, cbody,
                                   (jnp.int32(0), jnp.int32(0)))

        for cnt, ls_, ld_ in ((cnt2, l2s, l2d), (cnt3, l3s, l3d)):
            for j in range(_B // 16):
                idx = cnt + j * 16 + lax.iota(jnp.int32, 16)
                ridx = lax.shift_right_arithmetic(idx, 6)
                cidx = lax.bitwise_and(idx, _B - 1)
                plsc.store_scatter(ls_, [ridx, cidx],
                                   jnp.zeros((16,), jnp.int32),
                                   mask=jnp.full((16,), True))
                plsc.store_scatter(ld_, [ridx, cidx],
                                   jnp.full((16,), _N, jnp.int32),
                                   mask=jnp.full((16,), True))

        for cnt, cn_hbm in ((cnt2, cn2_hbm), (cnt3, cn3_hbm)):
            cbuf[pl.ds(0, 16)] = cnt + jnp.zeros((16,), jnp.int32)
            pltpu.sync_copy(cbuf, cn_hbm.at[wid])

        row0 = wid * _CAPR
        pltpu.sync_copy(l2s, c2s_hbm.at[pl.ds(row0, _CAPR)])
        pltpu.sync_copy(l2d, c2d_hbm.at[pl.ds(row0, _CAPR)])
        pltpu.sync_copy(l3s, c3s_hbm.at[pl.ds(row0, _CAPR)])
        pltpu.sync_copy(l3d, c3d_hbm.at[pl.ds(row0, _CAPR)])

    ltype = jax.ShapeDtypeStruct((nw * _CAPR, _B), jnp.int32)
    ctype = jax.ShapeDtypeStruct((nw, 16), jnp.int32)
    return pl.kernel(
        body,
        out_type=[ltype, ltype, ltype, ltype, ctype, ctype],
        mesh=mesh,
        compiler_params=pltpu.CompilerParams(
            needs_layout_passes=False, use_tc_tiling_on_sc=False),
        scratch_types=[
            pltpu.VMEM((ng, _B), jnp.int32),
            pltpu.VMEM((ng * _B, 128), jnp.int32),
            pltpu.VMEM((ng * _B, 128), jnp.int32),
            pltpu.VMEM((ng * _B, 128), jnp.int32),
            pltpu.VMEM((_CAPR, _B), jnp.int32),
            pltpu.VMEM((_CAPR, _B), jnp.int32),
            pltpu.VMEM((_CAPR, _B), jnp.int32),
            pltpu.VMEM((_CAPR, _B), jnp.int32),
            pltpu.VMEM((16,), jnp.int32),
            pltpu.SemaphoreType.DMA,
        ],
    )


def _sc_pass(n_pad, d, nc, ns):
    nw = nc * ns
    rows_pt = n_pad // ns
    mesh = plsc.VectorSubcoreMesh(core_axis_name="c", subcore_axis_name="s")

    def body(h_hbm, cls_hbm, cld_hbm, cnt_hbm, out_hbm,
             lidx, sbuf, dbuf, cbuf, rows_a, rows_b, zbuf, acc,
             sem, sem_a, sem_b):
        cid = lax.axis_index("c")
        sid = lax.axis_index("s")
        wid = cid * ns + sid
        row0 = sid * rows_pt

        for k in range((_CAPR // _B) * 4):
            lidx[k // 4, pl.ds((k % 4) * 16, 16)] = (
                wid * _CAPR + k * 16 + lax.iota(jnp.int32, 16))
        stage = []
        for g in range(_CAPR // _B):
            stage.append(pltpu.async_copy(
                cls_hbm.at[lidx.at[g]], sbuf.at[pl.ds(g * _B, _B)], sem))
            stage.append(pltpu.async_copy(
                cld_hbm.at[lidx.at[g]], dbuf.at[pl.ds(g * _B, _B)], sem))

        def zfill(r, carry):
            for j in range(d // 16):
                zbuf[r, pl.ds(j * 16, 16)] = jnp.zeros((16,), jnp.float32)
            return carry
        lax.fori_loop(0, _ZROWS, zfill, 0)
        for t in range(rows_pt // _ZROWS):
            pltpu.sync_copy(zbuf, acc.at[pl.ds(row0 + t * _ZROWS, _ZROWS)])
        zrem = rows_pt % _ZROWS
        if zrem:
            pltpu.sync_copy(
                zbuf.at[pl.ds(0, zrem)],
                acc.at[pl.ds(row0 + rows_pt - zrem, zrem)])

        for c in stage:
            c.wait()
        pltpu.sync_copy(cnt_hbm.at[wid], cbuf)
        cnt = cbuf[pl.ds(0, 16)][0]

        plsc.subcore_barrier()

        nb = (cnt + (_B - 1)) // _B

        def gs(i, carry):
            pltpu.async_copy(h_hbm.at[sbuf.at[i]], rows_a, sem_a).wait()
            pltpu.sync_copy(rows_a, acc.at[dbuf.at[i]], add=True)
            return carry

        lax.fori_loop(0, nb, gs, 0)

        plsc.subcore_barrier()

        pltpu.sync_copy(acc.at[pl.ds(row0, rows_pt)],
                        out_hbm.at[cid, pl.ds(row0, rows_pt)])

    return pl.kernel(
        body,
        out_type=jax.ShapeDtypeStruct((nc, n_pad, d), jnp.float32),
        mesh=mesh,
        compiler_params=pltpu.CompilerParams(
            needs_layout_passes=False, use_tc_tiling_on_sc=False),
        scratch_types=[
            pltpu.VMEM((_CAPR // _B, _B), jnp.int32),
            pltpu.VMEM((_CAPR, _B), jnp.int32),
            pltpu.VMEM((_CAPR, _B), jnp.int32),
            pltpu.VMEM((16,), jnp.int32),
            pltpu.VMEM((_B, d), jnp.float32),
            pltpu.VMEM((_B, d), jnp.float32),
            pltpu.VMEM((_ZROWS, d), jnp.float32),
            pltpu.VMEM_SHARED((n_pad, d), jnp.float32),
            pltpu.SemaphoreType.DMA,
            pltpu.SemaphoreType.DMA,
            pltpu.SemaphoreType.DMA,
        ],
    )


def _bcast(shape):
    return pl.BlockSpec(shape, lambda i: tuple(0 for _ in shape))


_R = 2000


def _tc1(x, w1, b1, w2, b2, w3, b3, lw0, lb0, lw1, lb1):
    n, d_in = x.shape
    hd = w1.shape[1]
    d2 = w3.shape[1]

    def body(x_r, w1_r, b1_r, w2_r, b2_r, w3_r, b3_r, lw0_r, lb0_r, lw1_r,
             lb1_r, h_r, hw0_r):
        h = jnp.maximum(
            jnp.dot(x_r[...], w1_r[...], preferred_element_type=jnp.float32)
            + b1_r[...], 0.0)
        h = jnp.maximum(
            jnp.dot(h, w2_r[...], preferred_element_type=jnp.float32)
            + b2_r[...], 0.0)
        h = (jnp.dot(h, w3_r[...], preferred_element_type=jnp.float32)
             + b3_r[...])
        h_r[...] = h
        hw0_r[...] = (jnp.dot(h, lw0_r[...] + lw1_r[...],
                              preferred_element_type=jnp.float32)
                      + lb0_r[...] + lb1_r[...])

    return pl.pallas_call(
        body,
        grid=(n // _R,),
        in_specs=[
            pl.BlockSpec((_R, d_in), lambda i: (i, 0)),
            _bcast((d_in, hd)), _bcast((1, hd)),
            _bcast((hd, hd)), _bcast((1, hd)),
            _bcast((hd, d2)), _bcast((1, d2)),
            _bcast((d2, hd)), _bcast((1, hd)),
            _bcast((d2, hd)), _bcast((1, hd)),
        ],
        out_specs=[
            pl.BlockSpec((_R, d2), lambda i: (i, 0)),
            pl.BlockSpec((_R, hd), lambda i: (i, 0)),
        ],
        out_shape=[
            jax.ShapeDtypeStruct((n, d2), jnp.float32),
            jax.ShapeDtypeStruct((n, hd), jnp.float32),
        ],
    )(x, w1, b1, w2, b2, w3, b3, lw0, lb0, lw1, lb1)


def _tc2(p0, hw0, wl, bl, w0, b0, w1, b1):
    nc, n_pad, d2 = p0.shape
    n, hd = hw0.shape

    def body(p_r, hw0_r, wl_r, bl_r, w0_r, b0_r, w1_r, b1_r, emb_r, e1w_r):
        aggr = p_r[0] + p_r[1]
        emb = jnp.maximum(
            jnp.dot(aggr, wl_r[...], preferred_element_type=jnp.float32)
            + bl_r[...] + hw0_r[...], 0.0)
        emb_r[...] = emb
        e1w_r[...] = (jnp.dot(emb, w0_r[...] + w1_r[...],
                              preferred_element_type=jnp.float32)
                      + b0_r[...] + b1_r[...])

    return pl.pallas_call(
        body,
        grid=(n // _R,),
        in_specs=[
            pl.BlockSpec((nc, _R, d2), lambda i: (0, i, 0)),
            pl.BlockSpec((_R, hd), lambda i: (i, 0)),
            _bcast((d2, hd)), _bcast((1, hd)),
            _bcast((hd, hd)), _bcast((1, hd)),
            _bcast((hd, hd)), _bcast((1, hd)),
        ],
        out_specs=[
            pl.BlockSpec((_R, hd), lambda i: (i, 0)),
            pl.BlockSpec((_R, hd), lambda i: (i, 0)),
        ],
        out_shape=[
            jax.ShapeDtypeStruct((n, hd), jnp.float32),
            jax.ShapeDtypeStruct((n, hd), jnp.float32),
        ],
    )(p0, hw0, wl, bl, w0, b0, w1, b1)


def _tc3(p1, e1w, wl, bl, fc1_w, fc1_b, fc2_w, fc2_b):
    nc, n_pad, hd = p1.shape
    n = e1w.shape[0]
    d_out = fc2_w.shape[1]

    def body(p_r, e1w_r, wl_r, bl_r, fc1w_r, fc1b_r, fc2w_r, fc2b_r, o_r):
        aggr = p_r[0] + p_r[1]
        emb = jnp.maximum(
            jnp.dot(aggr, wl_r[...], preferred_element_type=jnp.float32)
            + bl_r[...] + e1w_r[...], 0.0)
        fc1c = fc1w_r[0:hd, :] + fc1w_r[hd:2 * hd, :]
        t = jnp.maximum(
            jnp.dot(emb, fc1c, preferred_element_type=jnp.float32)
            + fc1b_r[...], 0.0)
        o = (jnp.dot(t, fc2w_r[...], preferred_element_type=jnp.float32)
             + fc2b_r[...])
        m = jnp.max(o, axis=1, keepdims=True)
        z = o - m
        o_r[...] = z - jnp.log(jnp.sum(jnp.exp(z), axis=1, keepdims=True))

    return pl.pallas_call(
        body,
        grid=(n // _R,),
        in_specs=[
            pl.BlockSpec((nc, _R, hd), lambda i: (0, i, 0)),
            pl.BlockSpec((_R, hd), lambda i: (i, 0)),
            _bcast((hd, hd)), _bcast((1, hd)),
            _bcast((2 * hd, hd)), _bcast((1, hd)),
            _bcast((hd, d_out)), _bcast((1, d_out)),
        ],
        out_specs=pl.BlockSpec((_R, d_out), lambda i: (i, 0)),
        out_shape=jax.ShapeDtypeStruct((n, d_out), jnp.float32),
    )(p1, e1w, wl, bl, fc1_w, fc1_b, fc2_w, fc2_b)


def kernel(x, edge_index, edge_type,
           mlp_w1, mlp_b1, mlp_w2, mlp_b2, mlp_w3, mlp_b3,
           l0_w0, l0_b0, l0_wl, l0_bl, l0_w1, l0_b1,
           l1_w0, l1_b0, l1_wl, l1_bl, l1_w1, l1_b1,
           fc1_w, fc1_b, fc2_w, fc2_b):
    e = edge_type.shape[0]
    info = plsc.get_sparse_core_info()
    nc, ns = info.num_cores, info.num_subcores
    nw = nc * ns

    erows = -(-e // (128 * nw)) * nw
    epad = erows * 128 - e
    se = jnp.pad(edge_index[1], (0, epad)).reshape(erows, 128)
    de = jnp.pad(edge_index[0], (0, epad)).reshape(erows, 128)
    te = jnp.pad(edge_type, (0, epad),
                 constant_values=-1).reshape(erows, 128)

    r2 = lambda b: b.reshape(1, -1)

    cl2s, cl2d, cl3s, cl3d, cn2, cn3 = _sc_compact(erows, nc, ns)(se, de, te)
    h, hw0 = _tc1(x, mlp_w1, r2(mlp_b1), mlp_w2, r2(mlp_b2),
                  mlp_w3, r2(mlp_b3), l0_w0, r2(l0_b0), l0_w1, r2(l0_b1))

    p0 = _sc_pass(_NPAD, h.shape[1], nc, ns)(h, cl2s, cl2d, cn2)
    emb1, e1w = _tc2(p0, hw0, l0_wl, r2(l0_bl),
                     l1_w0, r2(l1_b0), l1_w1, r2(l1_b1))
    p1 = _sc_pass(_NPAD, emb1.shape[1], nc, ns)(emb1, cl3s, cl3d, cn3)
    return _tc3(p1, e1w, l1_wl, r2(l1_bl), fc1_w, r2(fc1_b),
                fc2_w, r2(fc2_b))
